# Initial kernel scaffold; baseline (speedup 1.0000x reference)
import jax
import jax.numpy as jnp
from jax.experimental import pallas as pl


def kernel(x, Wg, W1, b1, W2, b2):
    raise NotImplementedError("write your pallas kernel here")


# speedup vs baseline: 3.7394x; 3.7394x over previous
import functools

import jax
import jax.numpy as jnp
from jax import lax
from jax.experimental import pallas as pl
from jax.experimental.pallas import tpu as pltpu
from jax.experimental.pallas import tpu_sc as plsc

S, H, E, I = 2048, 768, 64, 3072
T = 32
PADS = S + E * T
IB = 768
J = I // IB
NC, NS = 2, 16
NW = NC * NS
CHUNK = S // NW


def _route_body(x_ref, wg_ref, dest_ref, offs_ref):
    x = x_ref[...]
    wg = wg_ref[...]
    logits = lax.dot_general(x, wg, (((1,), (1,)), ((), ())),
                             preferred_element_type=jnp.float32)
    rowmax = jnp.max(logits, axis=1, keepdims=True)
    eiota = lax.broadcasted_iota(jnp.int32, (S, E), 1).astype(jnp.float32)
    ind = jnp.min(jnp.where(logits == rowmax, eiota, float(E)),
                  axis=1, keepdims=True)
    onehot = (eiota == ind).astype(jnp.float32)

    C = 256
    tri = (lax.broadcasted_iota(jnp.int32, (C, C), 0)
           > lax.broadcasted_iota(jnp.int32, (C, C), 1)).astype(jnp.float32)
    run = jnp.zeros((1, E), jnp.float32)
    ranks = []
    for c in range(S // C):
        oh = onehot[c * C:(c + 1) * C, :]
        ranks.append(lax.dot_general(tri, oh, (((1,), (0,)), ((), ())),
                                     preferred_element_type=jnp.float32) + run)
        run = run + jnp.sum(oh, axis=0, keepdims=True)
    rank = jnp.concatenate(ranks, axis=0)

    regions = jnp.floor((run + (T - 1)) * (1.0 / T)) * T
    tri_e = (lax.broadcasted_iota(jnp.int32, (E, E), 0)
             < lax.broadcasted_iota(jnp.int32, (E, E), 1)).astype(jnp.float32)
    offs = lax.dot_general(regions, tri_e, (((1,), (0,)), ((), ())),
                           preferred_element_type=jnp.float32)
    total = jnp.sum(regions, axis=1, keepdims=True)
    offs_full = jnp.concatenate([offs, jnp.broadcast_to(total, (1, E))], axis=1)

    dest = jnp.sum(onehot * (rank + offs), axis=1)
    dest_ref[...] = dest.astype(jnp.int32)
    offs_ref[...] = offs_full[0].astype(jnp.int32)


_route = pl.pallas_call(
    _route_body,
    out_shape=(jax.ShapeDtypeStruct((S,), jnp.int32),
               jax.ShapeDtypeStruct((2 * E,), jnp.int32)),
)


def _ffn_body(offs_ref, x_ref, w1_ref, b1_ref, w2_ref, b2_ref, out_ref):
    e = pl.program_id(0)
    j = pl.program_id(1)
    start = offs_ref[e]
    ntiles = (offs_ref[e + 1] - start) // T
    w1 = w1_ref[0]
    b1v = b1_ref[0, 0]
    w2 = w2_ref[0]
    b2v = b2_ref[0]

    def tile(i, carry):
        s = pl.multiple_of(start + i * T, T)
        xt = x_ref[pl.ds(s, T), :]
        h = lax.dot_general(xt, w1, (((1,), (1,)), ((), ())),
                            preferred_element_type=jnp.float32) + b1v
        h = h * (1.0 / (1.0 + jnp.exp(-h)))
        p = lax.dot_general(h, w2, (((1,), (1,)), ((), ())),
                            preferred_element_type=jnp.float32)

        @pl.when(j == 0)
        def _():
            out_ref[pl.ds(s, T), :] = p + b2v

        @pl.when(j != 0)
        def _():
            out_ref[pl.ds(s, T), :] = out_ref[pl.ds(s, T), :] + p

        return carry

    lax.fori_loop(0, ntiles, tile, 0)


_ffn = pl.pallas_call(
    _ffn_body,
    grid_spec=pltpu.PrefetchScalarGridSpec(
        num_scalar_prefetch=1,
        grid=(E, J),
        in_specs=[
            pl.BlockSpec(memory_space=pltpu.VMEM),
            pl.BlockSpec((1, IB, H), lambda e, j, offs: (e, j, 0)),
            pl.BlockSpec((1, 1, 1, IB), lambda e, j, offs: (e, j, 0, 0)),
            pl.BlockSpec((1, H, IB), lambda e, j, offs: (e, 0, j)),
            pl.BlockSpec((1, 1, H), lambda e, j, offs: (e, 0, 0)),
        ],
        out_specs=pl.BlockSpec(memory_space=pltpu.VMEM),
    ),
    out_shape=jax.ShapeDtypeStruct((PADS, H), jnp.float32),
)


@functools.cache
def _sc_kernels():
    scratch = [pltpu.VMEM((CHUNK,), jnp.int32),
               pltpu.VMEM((CHUNK, H), jnp.float32),
               pltpu.SemaphoreType.DMA]

    @functools.partial(
        pl.kernel,
        out_type=jax.ShapeDtypeStruct((PADS, H), jnp.float32),
        mesh=plsc.VectorSubcoreMesh(core_axis_name="c", subcore_axis_name="s",
                                    num_cores=NC, num_subcores=NS),
        scratch_types=scratch,
    )
    def sc_scatter(x_hbm, dest_hbm, out_hbm, idx_v, rows_v, sem):
        wid = lax.axis_index("s") * NC + lax.axis_index("c")
        base = wid * CHUNK
        pltpu.sync_copy(dest_hbm.at[pl.ds(base, CHUNK)], idx_v)
        pltpu.sync_copy(x_hbm.at[pl.ds(base, CHUNK)], rows_v)
        pltpu.async_copy(rows_v, out_hbm.at[idx_v], sem).wait()

    @functools.partial(
        pl.kernel,
        out_type=jax.ShapeDtypeStruct((S, H), jnp.float32),
        mesh=plsc.VectorSubcoreMesh(core_axis_name="c", subcore_axis_name="s",
                                    num_cores=NC, num_subcores=NS),
        scratch_types=scratch,
    )
    def sc_gather(outs_hbm, dest_hbm, out_hbm, idx_v, rows_v, sem):
        wid = lax.axis_index("s") * NC + lax.axis_index("c")
        base = wid * CHUNK
        pltpu.sync_copy(dest_hbm.at[pl.ds(base, CHUNK)], idx_v)
        pltpu.async_copy(outs_hbm.at[idx_v], rows_v, sem).wait()
        pltpu.sync_copy(rows_v, out_hbm.at[pl.ds(base, CHUNK)])

    return sc_scatter, sc_gather


def kernel(x, Wg, W1, b1, W2, b2):
    sc_scatter, sc_gather = _sc_kernels()
    x_flat = x.reshape(S, H)
    dest, offs = _route(x_flat, Wg)
    x_sorted = sc_scatter(x_flat, dest)
    out_sorted = _ffn(offs, x_sorted, W1, b1.reshape(E, J, 1, IB),
                      W2, b2.reshape(E, 1, H))
    out_flat = sc_gather(out_sorted, dest)
    return out_flat.reshape(x.shape)


# speedup vs baseline: 4.3743x; 1.1698x over previous
import functools

import jax
import jax.numpy as jnp
from jax import lax
from jax.experimental import pallas as pl
from jax.experimental.pallas import tpu as pltpu
from jax.experimental.pallas import tpu_sc as plsc

S, H, E, I = 2048, 768, 64, 3072
T = 32
PADS = S + E * T
IB = 1536
J = I // IB
NC, NS = 2, 16
NW = NC * NS
CHUNK = S // NW


def _route_body(x_ref, wg_ref, dest_ref, offs_ref):
    x = x_ref[...]
    wg = wg_ref[...]
    logits = lax.dot_general(x, wg, (((1,), (1,)), ((), ())),
                             preferred_element_type=jnp.float32)
    rowmax = jnp.max(logits, axis=1, keepdims=True)
    eiota = lax.broadcasted_iota(jnp.int32, (S, E), 1).astype(jnp.float32)
    ind = jnp.min(jnp.where(logits == rowmax, eiota, float(E)),
                  axis=1, keepdims=True)
    onehot = (eiota == ind).astype(jnp.float32)

    C = 256
    tri = (lax.broadcasted_iota(jnp.int32, (C, C), 0)
           > lax.broadcasted_iota(jnp.int32, (C, C), 1)).astype(jnp.float32)
    run = jnp.zeros((1, E), jnp.float32)
    ranks = []
    for c in range(S // C):
        oh = onehot[c * C:(c + 1) * C, :]
        ranks.append(lax.dot_general(tri, oh, (((1,), (0,)), ((), ())),
                                     preferred_element_type=jnp.float32) + run)
        run = run + jnp.sum(oh, axis=0, keepdims=True)
    rank = jnp.concatenate(ranks, axis=0)

    regions = jnp.floor((run + (T - 1)) * (1.0 / T)) * T
    tri_e = (lax.broadcasted_iota(jnp.int32, (E, E), 0)
             < lax.broadcasted_iota(jnp.int32, (E, E), 1)).astype(jnp.float32)
    offs = lax.dot_general(regions, tri_e, (((1,), (0,)), ((), ())),
                           preferred_element_type=jnp.float32)
    total = jnp.sum(regions, axis=1, keepdims=True)
    offs_full = jnp.concatenate([offs, jnp.broadcast_to(total, (1, E))], axis=1)

    dest = jnp.sum(onehot * (rank + offs), axis=1)
    dest_ref[...] = dest.astype(jnp.int32)
    offs_ref[...] = offs_full[0].astype(jnp.int32)


_route = pl.pallas_call(
    _route_body,
    out_shape=(jax.ShapeDtypeStruct((S,), jnp.int32),
               jax.ShapeDtypeStruct((2 * E,), jnp.int32)),
)


def _ffn_body(offs_ref, x_ref, w1_ref, b1_ref, w2_ref, b2_ref, out_ref):
    e = pl.program_id(0)
    j = pl.program_id(1)
    start = offs_ref[e]
    ntiles = (offs_ref[e + 1] - start) // T
    w1 = w1_ref[0]
    b1v = b1_ref[0, 0]
    w2 = w2_ref[0]
    b2v = b2_ref[0]

    def tile(i, carry):
        s = pl.multiple_of(start + i * T, T)
        xt = x_ref[pl.ds(s, T), :]
        h = lax.dot_general(xt, w1, (((1,), (1,)), ((), ())),
                            preferred_element_type=jnp.float32) + b1v
        h = h * (1.0 / (1.0 + jnp.exp(-h)))
        p = lax.dot_general(h, w2, (((1,), (1,)), ((), ())),
                            preferred_element_type=jnp.float32)

        @pl.when(j == 0)
        def _():
            out_ref[pl.ds(s, T), :] = p + b2v

        @pl.when(j != 0)
        def _():
            out_ref[pl.ds(s, T), :] = out_ref[pl.ds(s, T), :] + p

        return carry

    lax.fori_loop(0, ntiles, tile, 0)


_ffn = pl.pallas_call(
    _ffn_body,
    grid_spec=pltpu.PrefetchScalarGridSpec(
        num_scalar_prefetch=1,
        grid=(E, J),
        in_specs=[
            pl.BlockSpec(memory_space=pltpu.VMEM),
            pl.BlockSpec((1, IB, H), lambda e, j, offs: (e, j, 0)),
            pl.BlockSpec((1, 1, 1, IB), lambda e, j, offs: (e, j, 0, 0)),
            pl.BlockSpec((1, H, IB), lambda e, j, offs: (e, 0, j)),
            pl.BlockSpec((1, 1, H), lambda e, j, offs: (e, 0, 0)),
        ],
        out_specs=pl.BlockSpec(memory_space=pltpu.VMEM),
    ),
    out_shape=jax.ShapeDtypeStruct((PADS, H), jnp.float32),
)


@functools.cache
def _sc_kernels():
    scratch = [pltpu.VMEM((CHUNK,), jnp.int32),
               pltpu.VMEM((CHUNK, H), jnp.float32),
               pltpu.SemaphoreType.DMA]

    @functools.partial(
        pl.kernel,
        out_type=jax.ShapeDtypeStruct((PADS, H), jnp.float32),
        mesh=plsc.VectorSubcoreMesh(core_axis_name="c", subcore_axis_name="s",
                                    num_cores=NC, num_subcores=NS),
        scratch_types=scratch,
    )
    def sc_scatter(x_hbm, dest_hbm, out_hbm, idx_v, rows_v, sem):
        wid = lax.axis_index("s") * NC + lax.axis_index("c")
        base = wid * CHUNK
        pltpu.sync_copy(dest_hbm.at[pl.ds(base, CHUNK)], idx_v)
        pltpu.sync_copy(x_hbm.at[pl.ds(base, CHUNK)], rows_v)
        pltpu.async_copy(rows_v, out_hbm.at[idx_v], sem).wait()

    @functools.partial(
        pl.kernel,
        out_type=jax.ShapeDtypeStruct((S, H), jnp.float32),
        mesh=plsc.VectorSubcoreMesh(core_axis_name="c", subcore_axis_name="s",
                                    num_cores=NC, num_subcores=NS),
        scratch_types=scratch,
    )
    def sc_gather(outs_hbm, dest_hbm, out_hbm, idx_v, rows_v, sem):
        wid = lax.axis_index("s") * NC + lax.axis_index("c")
        base = wid * CHUNK
        pltpu.sync_copy(dest_hbm.at[pl.ds(base, CHUNK)], idx_v)
        pltpu.async_copy(outs_hbm.at[idx_v], rows_v, sem).wait()
        pltpu.sync_copy(rows_v, out_hbm.at[pl.ds(base, CHUNK)])

    return sc_scatter, sc_gather


def kernel(x, Wg, W1, b1, W2, b2):
    sc_scatter, sc_gather = _sc_kernels()
    x_flat = x.reshape(S, H)
    dest, offs = _route(x_flat, Wg)
    x_sorted = sc_scatter(x_flat, dest)
    out_sorted = _ffn(offs, x_sorted, W1, b1.reshape(E, J, 1, IB),
                      W2, b2.reshape(E, 1, H))
    out_flat = sc_gather(out_sorted, dest)
    return out_flat.reshape(x.shape)
